# Initial kernel scaffold; baseline (speedup 1.0000x reference)
import jax
import jax.numpy as jnp
from jax.experimental import pallas as pl


def kernel(x, edge_index, edge_type, edge_emb, W1, att_src1, att_dst1, lin_edge1, att_edge1, b1, W2, att_src2, att_dst2, lin_edge2, att_edge2, b2):
    raise NotImplementedError("write your pallas kernel here")


# speedup vs baseline: 1.0556x; 1.0556x over previous
import jax
import jax.numpy as jnp
from jax.experimental import pallas as pl

N = 10000
E = 320000
D = 128
H = 8
C = 16
NEG_SLOPE = 0.2


def _gat_layer(xin, src, dst, edge_type, edge_emb, W, att_src, att_dst,
               lin_edge, att_edge, bias):
    h = (xin @ W).reshape(N, H, C)
    a_src = jnp.einsum('nhc,hc->nh', h, att_src)
    a_dst = jnp.einsum('nhc,hc->nh', h, att_dst)
    a_et = ((edge_emb @ lin_edge).reshape(-1, H, C) * att_edge).sum(-1)
    bnd = a_src.max(0) + a_dst.max(0) + a_et.max(0)
    bnd = jnp.where(bnd > 0, bnd, NEG_SLOPE * bnd)
    alpha = a_src[src] + a_dst[dst] + a_et[edge_type]
    alpha = jnp.where(alpha > 0, alpha, NEG_SLOPE * alpha)
    ex = jnp.exp(alpha - bnd[None, :])
    denom = jax.ops.segment_sum(ex, dst, num_segments=N)
    msg = h[src] * ex[:, :, None]
    accum = jax.ops.segment_sum(msg, dst, num_segments=N)
    out = jnp.where(denom[:, :, None] > 0, accum / denom[:, :, None], 0.0)
    return out.reshape(N, H * C) + bias[None, :]


def kernel(x, edge_index, edge_type, edge_emb, W1, att_src1, att_dst1,
           lin_edge1, att_edge1, b1, W2, att_src2, att_dst2, lin_edge2,
           att_edge2, b2):
    src, dst = edge_index[0], edge_index[1]
    h = _gat_layer(x, src, dst, edge_type, edge_emb, W1, att_src1, att_dst1,
                   lin_edge1, att_edge1, b1)
    h = jax.nn.relu(h)
    return _gat_layer(h, src, dst, edge_type, edge_emb, W2, att_src2,
                      att_dst2, lin_edge2, att_edge2, b2)


# speedup vs baseline: 38.4735x; 36.4455x over previous
import functools

import jax
import jax.numpy as jnp
from jax import lax
from jax.experimental import pallas as pl
from jax.experimental.pallas import tpu as pltpu
from jax.experimental.pallas import tpu_sc as plsc

N = 10000
E = 320000
D = 128
H = 8
C = 16
T = 16
NEG_SLOPE = 0.2

NC = 2
NS = 16
NW = NC * NS
EPT = E // NW
K = 80
NCHUNK = EPT // K
NPAD = 10240
RPT = NPAD // NS
AW = 144
LANES = 16


def _make_B(att):
    eye = jnp.concatenate(
        [jnp.eye(H, dtype=jnp.float32), jnp.zeros((H, LANES - H), jnp.float32)], axis=1)
    return (att[:, :, None] * eye[:, None, :]).reshape(H * C, LANES)


def _make_R():
    r = jnp.zeros((H, D), jnp.float32)
    idx = jnp.arange(D)
    return r.at[idx // C, idx].set(1.0)


def _proj_body(x_ref, w_ref, bs_ref, bd_ref, be_ref, emb_ref, lin_ref,
               haug_ref, adst_ref, bound_ref, aet_ref):
    h = jnp.dot(x_ref[...], w_ref[...], preferred_element_type=jnp.float32)
    asrc = jnp.dot(h, bs_ref[...], preferred_element_type=jnp.float32)
    adst = jnp.dot(h, bd_ref[...], preferred_element_type=jnp.float32)
    aet_full = jnp.dot(emb_ref[...], lin_ref[...], preferred_element_type=jnp.float32)
    aet = jnp.dot(aet_full, be_ref[...], preferred_element_type=jnp.float32)
    haug_ref[:, :D] = h
    haug_ref[:, D:] = asrc
    adst_ref[...] = adst
    bnd = (jnp.max(asrc, axis=0, keepdims=True)
           + jnp.max(adst, axis=0, keepdims=True)
           + jnp.max(aet, axis=0, keepdims=True))
    bound_ref[...] = jnp.where(bnd > 0, bnd, NEG_SLOPE * bnd)
    pad = jnp.concatenate(
        [jnp.zeros((1, H), jnp.float32), jnp.full((1, LANES - H), -1e30, jnp.float32)],
        axis=1)
    aet_ref[...] = aet + pad


_proj = pl.pallas_call(
    _proj_body,
    out_shape=[
        jax.ShapeDtypeStruct((N, AW), jnp.float32),
        jax.ShapeDtypeStruct((N, LANES), jnp.float32),
        jax.ShapeDtypeStruct((1, LANES), jnp.float32),
        jax.ShapeDtypeStruct((T, LANES), jnp.float32),
    ],
)


def _edge_body(haug_hbm, adst_hbm, aet_hbm, bound_hbm, src_hbm, dst_hbm, et_hbm,
               out_hbm, rows_v, msg_v, adst_v, srcidx_v, dstidx_v, et_v, aet_v,
               bound_v, acc_sh, sem1, sem2):
    cid = lax.axis_index("c")
    sid = lax.axis_index("s")
    wid = cid * NS + sid

    pltpu.sync_copy(aet_hbm, aet_v)
    pltpu.sync_copy(bound_hbm, bound_v)

    zero16 = jnp.zeros((LANES,), jnp.float32)

    def zrow(i, _):
        for j in range(AW // LANES):
            msg_v[i, pl.ds(j * LANES, LANES)] = zero16
        return 0

    lax.fori_loop(0, K, zrow, 0)

    base = sid * RPT
    for kc in range(RPT // K):
        pltpu.sync_copy(msg_v, acc_sh.at[pl.ds(base + kc * K, K)])
    plsc.subcore_barrier()

    iota16 = lax.iota(jnp.int32, LANES)
    ebase = wid * EPT

    def chunk(kc, _):
        cb = ebase + kc * K
        pltpu.sync_copy(src_hbm.at[pl.ds(cb, K)], srcidx_v)
        pltpu.sync_copy(dst_hbm.at[pl.ds(cb, K)], dstidx_v)
        pltpu.sync_copy(et_hbm.at[pl.ds(cb, K)], et_v)
        cp1 = pltpu.async_copy(haug_hbm.at[srcidx_v], rows_v, sem1)
        cp2 = pltpu.async_copy(adst_hbm.at[dstidx_v], adst_v, sem2)
        cp1.wait()
        cp2.wait()

        def edge(i, _):
            ivec = jnp.full((LANES,), i, jnp.int32)
            asrc = rows_v[i, pl.ds(D, LANES)]
            adst = adst_v[i]
            tvec = plsc.load_gather(et_v, [ivec])
            aet = plsc.load_gather(aet_v, [tvec, iota16])
            alpha = asrc + adst + aet
            alpha = jnp.maximum(alpha, NEG_SLOPE * alpha)
            ex = jnp.exp(alpha - bound_v[0])
            msg_v[i, pl.ds(D, LANES)] = ex
            for j in range(H):
                exj = plsc.load_gather(msg_v, [ivec, jnp.full((LANES,), D + j, jnp.int32)])
                msg_v[i, pl.ds(j * C, C)] = rows_v[i, pl.ds(j * C, C)] * exj
            return 0

        lax.fori_loop(0, K, edge, 0)
        pltpu.sync_copy(msg_v, acc_sh.at[dstidx_v], add=True)
        return 0

    lax.fori_loop(0, NCHUNK, chunk, 0)
    plsc.subcore_barrier()
    pltpu.sync_copy(acc_sh.at[pl.ds(sid * RPT, RPT)],
                    out_hbm.at[cid, pl.ds(sid * RPT, RPT)])


_edge = pl.kernel(
    _edge_body,
    out_type=jax.ShapeDtypeStruct((NC, NPAD, AW), jnp.float32),
    mesh=plsc.VectorSubcoreMesh(core_axis_name="c", subcore_axis_name="s"),
    compiler_params=pltpu.CompilerParams(needs_layout_passes=False,
                                         use_tc_tiling_on_sc=False),
    scratch_types=[
        pltpu.VMEM((K, AW), jnp.float32),
        pltpu.VMEM((K, AW), jnp.float32),
        pltpu.VMEM((K, LANES), jnp.float32),
        pltpu.VMEM((K,), jnp.int32),
        pltpu.VMEM((K,), jnp.int32),
        pltpu.VMEM((K,), jnp.int32),
        pltpu.VMEM((T, LANES), jnp.float32),
        pltpu.VMEM((1, LANES), jnp.float32),
        pltpu.VMEM_SHARED((NPAD, AW), jnp.float32),
        pltpu.SemaphoreType.DMA,
        pltpu.SemaphoreType.DMA,
    ],
)


def _fin_body(p_ref, bias_ref, r_ref, out_ref, *, relu):
    p = p_ref[0, :N] + p_ref[1, :N]
    accum = p[:, :D]
    denom = p[:, D:D + H]
    dinv = jnp.where(denom > 0, 1.0 / denom, 0.0)
    drep = jnp.dot(dinv, r_ref[...], preferred_element_type=jnp.float32)
    out = accum * drep + bias_ref[...]
    if relu:
        out = jnp.maximum(out, 0.0)
    out_ref[...] = out


def _fin(relu):
    return pl.pallas_call(
        functools.partial(_fin_body, relu=relu),
        out_shape=jax.ShapeDtypeStruct((N, D), jnp.float32),
    )


def kernel(x, edge_index, edge_type, edge_emb, W1, att_src1, att_dst1,
           lin_edge1, att_edge1, b1, W2, att_src2, att_dst2, lin_edge2,
           att_edge2, b2):
    src, dst = edge_index[0], edge_index[1]

    def layer(xin, W, att_src, att_dst, lin_edge, att_edge, bias, relu):
        haug, adst, bound, aet = _proj(xin, W, _make_B(att_src), _make_B(att_dst),
                                       _make_B(att_edge), edge_emb, lin_edge)
        part = _edge(haug, adst, aet, bound, src, dst, edge_type)
        return _fin(relu)(part, bias.reshape(1, D), _make_R())

    h1 = layer(x, W1, att_src1, att_dst1, lin_edge1, att_edge1, b1, True)
    return layer(h1, W2, att_src2, att_dst2, lin_edge2, att_edge2, b2, False)
